# baseline (speedup 1.0000x reference)
import functools

import jax
import jax.numpy as jnp
from jax import lax
from jax.experimental import pallas as pl
from jax.experimental.pallas import tpu as pltpu
from jax.experimental.pallas import tpu_sc as plsc

VOCAB = 1000000
D = 64
B = 4096
SEQ = 200
N = B * SEQ
EPS = 1e-5

NC, NS = 2, 16
NW = NC * NS
PER_W = N // NW
CH = 128
NCH = PER_W // CH

_mesh = plsc.VectorSubcoreMesh(core_axis_name="c", subcore_axis_name="s")


@functools.partial(
    pl.kernel,
    out_type=jax.ShapeDtypeStruct((N, D), jnp.float32),
    mesh=_mesh,
    scratch_types=[
        pltpu.VMEM((CH,), jnp.int32),
        pltpu.VMEM((CH, D), jnp.float32),
        pltpu.VMEM((D,), jnp.float32),
        pltpu.VMEM((D,), jnp.float32),
        pltpu.SemaphoreType.DMA,
    ],
    compiler_params=pltpu.CompilerParams(
        needs_layout_passes=False, use_tc_tiling_on_sc=False),
)
def _emb_ln(x_hbm, table_hbm, gamma_hbm, beta_hbm, out_hbm,
            idx_v, rows_v, gamma_v, beta_v, sem):
    wid = lax.axis_index("s") * NC + lax.axis_index("c")
    base0 = wid * PER_W

    pltpu.sync_copy(gamma_hbm, gamma_v)
    pltpu.sync_copy(beta_hbm, beta_v)
    gvs = [gamma_v[pl.ds(c * 16, 16)] for c in range(D // 16)]
    bvs = [beta_v[pl.ds(c * 16, 16)] for c in range(D // 16)]

    def chunk_body(ci, carry):
        base = base0 + ci * CH
        pltpu.sync_copy(x_hbm.at[pl.ds(base, CH)], idx_v)
        pltpu.async_copy(table_hbm.at[idx_v], rows_v, sem).wait()

        def row_body(r, c2):
            vs = [rows_v[r, pl.ds(c * 16, 16)] for c in range(D // 16)]
            s = (vs[0] + vs[1]) + (vs[2] + vs[3])
            sq = (vs[0] * vs[0] + vs[1] * vs[1]) + (vs[2] * vs[2] + vs[3] * vs[3])
            tot = jnp.sum(s)
            tot2 = jnp.sum(sq)
            mean = tot * (1.0 / D)
            var = tot2 * (1.0 / D) - mean * mean
            xv = var + EPS
            i = lax.bitcast_convert_type(xv, jnp.int32)
            i = jnp.int32(0x5F3759DF) - lax.shift_right_logical(i, 1)
            y = lax.bitcast_convert_type(i, jnp.float32)
            hx = 0.5 * xv
            y = y * (1.5 - hx * y * y)
            y = y * (1.5 - hx * y * y)
            y = y * (1.5 - hx * y * y)
            for c in range(D // 16):
                rows_v[r, pl.ds(c * 16, 16)] = (vs[c] - mean) * y * gvs[c] + bvs[c]
            return c2

        lax.fori_loop(0, CH, row_body, 0)
        pltpu.sync_copy(rows_v, out_hbm.at[pl.ds(base, CH)])
        return carry

    lax.fori_loop(0, NCH, chunk_body, 0)


def kernel(x, table, gamma, beta):
    xf = x.reshape(-1).astype(jnp.int32)
    out = _emb_ln(xf, table, gamma, beta)
    return out.reshape(x.shape + (D,))
